# baseline (speedup 1.0000x reference)
import functools

import jax
import jax.numpy as jnp
from jax import lax
from jax.experimental import pallas as pl
from jax.experimental.pallas import tpu as pltpu
from jax.experimental.pallas import tpu_sc as plsc

NC = 1
NS = 16
LANES = 16
CHUNK = 128
D = 128


def _make_edge_pass(n, npad, e_total, ch):
    t_per_w = ch * CHUNK
    rows_per_sub = npad // NS
    nfull = rows_per_sub // CHUNK
    rem = rows_per_sub - nfull * CHUNK
    nd = npad // CHUNK
    assert rows_per_sub % 8 == 0 and nd <= CHUNK
    mesh = plsc.VectorSubcoreMesh(
        core_axis_name="c", subcore_axis_name="s",
        num_cores=NC, num_subcores=NS)

    @functools.partial(
        pl.kernel,
        out_type=[jax.ShapeDtypeStruct((npad, D), jnp.float32),
                  jax.ShapeDtypeStruct((nd, CHUNK), jnp.float32)],
        mesh=mesh,
        compiler_params=pltpu.CompilerParams(needs_layout_passes=False),
        scratch_types=[
            pltpu.VMEM((CHUNK,), jnp.int32),
            pltpu.VMEM((CHUNK,), jnp.int32),
            pltpu.VMEM((CHUNK,), jnp.int32),
            pltpu.VMEM((CHUNK, D), jnp.float32),
            pltpu.VMEM((CHUNK, D), jnp.float32),
            pltpu.VMEM((D,), jnp.float32),
            pltpu.VMEM_SHARED((npad, D), jnp.float32),
            pltpu.VMEM_SHARED((nd, CHUNK), jnp.float32),
            pltpu.SemaphoreType.DMA,
            pltpu.SemaphoreType.DMA,
        ])
    def edge_kernel(xl_hbm, xr_hbm, src_hbm, dst_hbm, att_hbm,
                    num_hbm, den_hbm,
                    src_v, dst_v, dsth_v, rows_l, rows_r,
                    att_v, num_acc, den_acc, sem1, sem2):
        s = lax.axis_index("s")
        wid = s
        lane = jnp.arange(LANES, dtype=jnp.int32)
        zeros16 = jnp.zeros((LANES,), jnp.float32)

        pltpu.sync_copy(att_hbm, att_v)

        def zrow(r, carry):
            for i in range(D // LANES):
                rows_l[r, pl.ds(i * LANES, LANES)] = zeros16
            return carry
        lax.fori_loop(0, CHUNK, zrow, 0)
        row0 = pl.multiple_of(s * rows_per_sub, 8)
        for i in range(nfull):
            pltpu.sync_copy(rows_l.at[pl.ds(0, CHUNK)],
                            num_acc.at[pl.ds(row0 + i * CHUNK, CHUNK)])
        if rem:
            pltpu.sync_copy(rows_l.at[pl.ds(0, rem)],
                            num_acc.at[pl.ds(row0 + nfull * CHUNK, rem)])
        @pl.when(s == 0)
        def _():
            pltpu.sync_copy(rows_l.at[pl.ds(0, nd)], den_acc)
        plsc.subcore_barrier()

        eids = [lane + (g * LANES) for g in range(8)]

        def chunk_body(chi, carry):
            base = wid * t_per_w + chi * CHUNK
            pltpu.sync_copy(src_hbm.at[pl.ds(base, CHUNK)], src_v)
            pltpu.sync_copy(dst_hbm.at[pl.ds(base, CHUNK)], dst_v)
            cp_l = pltpu.async_copy(xl_hbm.at[src_v], rows_l, sem1)
            cp_r = pltpu.async_copy(xr_hbm.at[dst_v], rows_r, sem2)
            cp_l.wait()
            cp_r.wait()

            def kbody(k, accs):
                kv = jnp.full((LANES,), k, dtype=jnp.int32)
                ak = plsc.load_gather(att_v, [kv])
                out = []
                for g in range(8):
                    vl = plsc.load_gather(rows_l, [eids[g], kv])
                    vr = plsc.load_gather(rows_r, [eids[g], kv])
                    m = vl + vr
                    a = jnp.maximum(m, 0.2 * m)
                    out.append(accs[g] + ak * a)
                return tuple(out)
            accs = lax.fori_loop(0, D, kbody,
                                 tuple([zeros16] * 8), unroll=2)
            ps = []
            los = []
            for g in range(8):
                gid = base + (g * LANES) + lane
                ps.append(jnp.where(gid < e_total, jnp.exp(accs[g]), 0.0))
                dg = dst_v[pl.ds(g * LANES, LANES)]
                los.append(jnp.bitwise_and(dg, CHUNK - 1))
                dsth_v[pl.ds(g * LANES, LANES)] = jnp.right_shift(dg, 7)

            def zr(r, carry):
                for i in range(D // LANES):
                    rows_r[r, pl.ds(i * LANES, LANES)] = zeros16
                return carry
            lax.fori_loop(0, CHUNK, zr, 0)
            for g in range(8):
                plsc.store_scatter(rows_r, [eids[g], los[g]], ps[g])

            def wcol(k, carry):
                kv = jnp.full((LANES,), k, dtype=jnp.int32)
                for g in range(8):
                    vl = plsc.load_gather(rows_l, [eids[g], kv])
                    plsc.store_scatter(rows_l, [eids[g], kv],
                                       vl * ps[g])
                return carry
            lax.fori_loop(0, D, wcol, 0, unroll=2)
            pltpu.sync_copy(rows_l, num_acc.at[dst_v], add=True)
            pltpu.sync_copy(rows_r, den_acc.at[dsth_v], add=True)
            return carry
        lax.fori_loop(0, ch, chunk_body, 0)

        plsc.subcore_barrier()
        for i in range(nfull):
            pltpu.sync_copy(num_acc.at[pl.ds(row0 + i * CHUNK, CHUNK)],
                            num_hbm.at[pl.ds(row0 + i * CHUNK, CHUNK)])
        if rem:
            pltpu.sync_copy(num_acc.at[pl.ds(row0 + nfull * CHUNK, rem)],
                            num_hbm.at[pl.ds(row0 + nfull * CHUNK, rem)])
        @pl.when(s == 0)
        def _():
            pltpu.sync_copy(den_acc, den_hbm)

    return edge_kernel


def _mm4(x, w1, w2, w3, w4):
    n = x.shape[0]
    blk = 1000
    grid = n // blk

    def body(x_ref, w1_ref, w2_ref, w3_ref, w4_ref, o1, o2, o3, o4):
        xb = x_ref[...]
        for w_ref, o_ref in ((w1_ref, o1), (w2_ref, o2),
                             (w3_ref, o3), (w4_ref, o4)):
            o_ref[...] = lax.dot_general(
                xb, w_ref[...], (((1,), (1,)), ((), ())),
                preferred_element_type=jnp.float32,
                precision=lax.Precision.HIGHEST)

    wspec = pl.BlockSpec((D, D), lambda i: (0, 0))
    return pl.pallas_call(
        body,
        grid=(grid,),
        in_specs=[pl.BlockSpec((blk, D), lambda i: (i, 0))] + [wspec] * 4,
        out_specs=[pl.BlockSpec((blk, D), lambda i: (i, 0))] * 4,
        out_shape=[jax.ShapeDtypeStruct((n, D), jnp.float32)] * 4,
    )(x, w1, w2, w3, w4)


def _combine(num_ref, den_ref, b_ref, g_ref, be_ref):
    a = num_ref[...]
    den = den_ref[...]
    h = a / (den + 1e-16) + b_ref[...]
    mean = jnp.mean(h, axis=0, keepdims=True)
    var = jnp.mean((h - mean) ** 2, axis=0, keepdims=True)
    return (h - mean) * lax.rsqrt(var + 1e-5) * g_ref[...] + be_ref[...]


def _mid(num, den, b, g, be, wl, wr):
    n = num.shape[0]

    def body(num_ref, den_ref, b_ref, g_ref, be_ref, wl_ref, wr_ref,
             ol, orr):
        hn = jnp.maximum(_combine(num_ref, den_ref, b_ref, g_ref, be_ref),
                         0.0)
        for w_ref, o_ref in ((wl_ref, ol), (wr_ref, orr)):
            o_ref[...] = lax.dot_general(
                hn, w_ref[...], (((1,), (1,)), ((), ())),
                preferred_element_type=jnp.float32,
                precision=lax.Precision.HIGHEST)

    return pl.pallas_call(
        body,
        out_shape=[jax.ShapeDtypeStruct((n, D), jnp.float32)] * 2,
    )(num, den, b, g, be, wl, wr)


def _final(num2, den2, b2, g2, be2, num3, den3, b3, g3, be3):
    n = num2.shape[0]

    def body(n2, d2, b2r, g2r, be2r, n3, d3, b3r, g3r, be3r, out):
        o2 = _combine(n2, d2, b2r, g2r, be2r)
        o3 = _combine(n3, d3, b3r, g3r, be3r)
        out[...] = jnp.maximum(o2 + o3, 0.0)

    return pl.pallas_call(
        body,
        out_shape=jax.ShapeDtypeStruct((n, D), jnp.float32),
    )(num2, den2, b2, g2, be2, num3, den3, b3, g3, be3)


def kernel(x, edge_index, Wl1, Wr1, att1, b1, g1, be1,
           Wl2, Wr2, att2, b2, g2, be2,
           Wl3, Wr3, att3, b3, g3, be3):
    n = x.shape[0]
    e = edge_index.shape[1]
    e_total = e + n
    ch = -(-e_total // (NC * NS * CHUNK))
    e_pad = NC * NS * CHUNK * ch

    idt = edge_index.dtype
    loop = jnp.arange(n, dtype=idt)
    padz = jnp.zeros((e_pad - e_total,), idt)
    src = jnp.concatenate([edge_index[0], loop, padz]).astype(jnp.int32)
    dst = jnp.concatenate([edge_index[1], loop, padz]).astype(jnp.int32)

    npad = -(-n // (NS * 8)) * (NS * 8)
    edge_pass = _make_edge_pass(n, npad, e_total, ch)

    def split(acc):
        num, den = acc
        return num[:n], den.reshape(npad)[:n].reshape(n, 1)

    def row(v):
        return v.reshape(1, D)

    xl1, xr1, xl3, xr3 = _mm4(x, Wl1, Wr1, Wl3, Wr3)
    num1, den1 = split(edge_pass(xl1, xr1, src, dst, att1))
    xl2, xr2 = _mid(num1, den1, row(b1), row(g1), row(be1), Wl2, Wr2)
    num2, den2 = split(edge_pass(xl2, xr2, src, dst, att2))
    num3, den3 = split(edge_pass(xl3, xr3, src, dst, att3))
    return _final(num2, den2, row(b2), row(g2), row(be2),
                  num3, den3, row(b3), row(g3), row(be3))


# speedup vs baseline: 1.9598x; 1.9598x over previous
import functools

import jax
import jax.numpy as jnp
from jax import lax
from jax.experimental import pallas as pl
from jax.experimental.pallas import tpu as pltpu
from jax.experimental.pallas import tpu_sc as plsc

NC = 2
NS = 16
LANES = 16
CHUNK = 128
D = 128


def _make_edge_pass(n, npad, e_total, ch):
    t_per_w = ch * CHUNK
    rows_per_sub = npad // NS
    nfull = rows_per_sub // CHUNK
    rem = rows_per_sub - nfull * CHUNK
    nd = npad // CHUNK
    assert rows_per_sub % 8 == 0 and nd <= CHUNK
    mesh = plsc.VectorSubcoreMesh(
        core_axis_name="c", subcore_axis_name="s",
        num_cores=NC, num_subcores=NS)

    @functools.partial(
        pl.kernel,
        out_type=[jax.ShapeDtypeStruct((NC, npad, D), jnp.float32),
                  jax.ShapeDtypeStruct((NC, nd, CHUNK), jnp.float32)],
        mesh=mesh,
        compiler_params=pltpu.CompilerParams(needs_layout_passes=False),
        scratch_types=[
            pltpu.VMEM((CHUNK,), jnp.int32),
            pltpu.VMEM((CHUNK,), jnp.int32),
            pltpu.VMEM((CHUNK,), jnp.int32),
            pltpu.VMEM((CHUNK, D), jnp.float32),
            pltpu.VMEM((CHUNK, D), jnp.float32),
            pltpu.VMEM((D,), jnp.float32),
            pltpu.VMEM_SHARED((npad, D), jnp.float32),
            pltpu.VMEM_SHARED((nd, CHUNK), jnp.float32),
            pltpu.SemaphoreType.DMA,
            pltpu.SemaphoreType.DMA,
        ])
    def edge_kernel(xl_hbm, xr_hbm, src_hbm, dst_hbm, att_hbm,
                    num_hbm, den_hbm,
                    src_v, dst_v, dsth_v, rows_l, rows_r,
                    att_v, num_acc, den_acc, sem1, sem2):
        c = lax.axis_index("c")
        s = lax.axis_index("s")
        wid = c * NS + s
        lane = jnp.arange(LANES, dtype=jnp.int32)
        zeros16 = jnp.zeros((LANES,), jnp.float32)

        pltpu.sync_copy(att_hbm, att_v)

        def zrow(r, carry):
            for i in range(D // LANES):
                rows_l[r, pl.ds(i * LANES, LANES)] = zeros16
            return carry
        lax.fori_loop(0, CHUNK, zrow, 0)
        row0 = pl.multiple_of(s * rows_per_sub, 8)
        for i in range(nfull):
            pltpu.sync_copy(rows_l.at[pl.ds(0, CHUNK)],
                            num_acc.at[pl.ds(row0 + i * CHUNK, CHUNK)])
        if rem:
            pltpu.sync_copy(rows_l.at[pl.ds(0, rem)],
                            num_acc.at[pl.ds(row0 + nfull * CHUNK, rem)])
        @pl.when(s == 0)
        def _():
            pltpu.sync_copy(rows_l.at[pl.ds(0, nd)], den_acc)
        plsc.subcore_barrier()

        eids = [lane + (g * LANES) for g in range(8)]

        def chunk_body(chi, carry):
            base = wid * t_per_w + chi * CHUNK
            pltpu.sync_copy(src_hbm.at[pl.ds(base, CHUNK)], src_v)
            pltpu.sync_copy(dst_hbm.at[pl.ds(base, CHUNK)], dst_v)
            cp_l = pltpu.async_copy(xl_hbm.at[src_v], rows_l, sem1)
            cp_r = pltpu.async_copy(xr_hbm.at[dst_v], rows_r, sem2)
            cp_l.wait()
            cp_r.wait()

            def kbody(k, accs):
                kv = jnp.full((LANES,), k, dtype=jnp.int32)
                ak = plsc.load_gather(att_v, [kv])
                out = []
                for g in range(8):
                    vl = plsc.load_gather(rows_l, [eids[g], kv])
                    vr = plsc.load_gather(rows_r, [eids[g], kv])
                    m = vl + vr
                    a = jnp.maximum(m, 0.2 * m)
                    out.append(accs[g] + ak * a)
                return tuple(out)
            accs = lax.fori_loop(0, D, kbody,
                                 tuple([zeros16] * 8), unroll=2)
            ps = []
            los = []
            for g in range(8):
                gid = base + (g * LANES) + lane
                ps.append(jnp.where(gid < e_total, jnp.exp(accs[g]), 0.0))
                dg = dst_v[pl.ds(g * LANES, LANES)]
                los.append(jnp.bitwise_and(dg, CHUNK - 1))
                dsth_v[pl.ds(g * LANES, LANES)] = jnp.right_shift(dg, 7)

            def zr(r, carry):
                for i in range(D // LANES):
                    rows_r[r, pl.ds(i * LANES, LANES)] = zeros16
                return carry
            lax.fori_loop(0, CHUNK, zr, 0)
            for g in range(8):
                plsc.store_scatter(rows_r, [eids[g], los[g]], ps[g])

            def wcol(k, carry):
                kv = jnp.full((LANES,), k, dtype=jnp.int32)
                for g in range(8):
                    vl = plsc.load_gather(rows_l, [eids[g], kv])
                    plsc.store_scatter(rows_l, [eids[g], kv],
                                       vl * ps[g])
                return carry
            lax.fori_loop(0, D, wcol, 0, unroll=2)
            pltpu.sync_copy(rows_l, num_acc.at[dst_v], add=True)
            pltpu.sync_copy(rows_r, den_acc.at[dsth_v], add=True)
            return carry
        lax.fori_loop(0, ch, chunk_body, 0)

        plsc.subcore_barrier()
        for i in range(nfull):
            pltpu.sync_copy(num_acc.at[pl.ds(row0 + i * CHUNK, CHUNK)],
                            num_hbm.at[c, pl.ds(row0 + i * CHUNK, CHUNK)])
        if rem:
            pltpu.sync_copy(num_acc.at[pl.ds(row0 + nfull * CHUNK, rem)],
                            num_hbm.at[c, pl.ds(row0 + nfull * CHUNK, rem)])
        @pl.when(s == 0)
        def _():
            pltpu.sync_copy(den_acc, den_hbm.at[c])

    return edge_kernel


def _mm4(x, w1, w2, w3, w4):
    n = x.shape[0]
    blk = 1000
    grid = n // blk

    def body(x_ref, w1_ref, w2_ref, w3_ref, w4_ref, o1, o2, o3, o4):
        xb = x_ref[...]
        for w_ref, o_ref in ((w1_ref, o1), (w2_ref, o2),
                             (w3_ref, o3), (w4_ref, o4)):
            o_ref[...] = lax.dot_general(
                xb, w_ref[...], (((1,), (1,)), ((), ())),
                preferred_element_type=jnp.float32,
                precision=lax.Precision.HIGHEST)

    wspec = pl.BlockSpec((D, D), lambda i: (0, 0))
    return pl.pallas_call(
        body,
        grid=(grid,),
        in_specs=[pl.BlockSpec((blk, D), lambda i: (i, 0))] + [wspec] * 4,
        out_specs=[pl.BlockSpec((blk, D), lambda i: (i, 0))] * 4,
        out_shape=[jax.ShapeDtypeStruct((n, D), jnp.float32)] * 4,
    )(x, w1, w2, w3, w4)


def _combine(num_ref, den_ref, b_ref, g_ref, be_ref):
    a = num_ref[0]
    den = den_ref[:, 0:1]
    for i in range(1, NC):
        a = a + num_ref[i]
        den = den + den_ref[:, i:i + 1]
    h = a / (den + 1e-16) + b_ref[...]
    mean = jnp.mean(h, axis=0, keepdims=True)
    var = jnp.mean((h - mean) ** 2, axis=0, keepdims=True)
    return (h - mean) * lax.rsqrt(var + 1e-5) * g_ref[...] + be_ref[...]


def _mid(num, den, b, g, be, wl, wr):
    n = num.shape[1]

    def body(num_ref, den_ref, b_ref, g_ref, be_ref, wl_ref, wr_ref,
             ol, orr):
        hn = jnp.maximum(_combine(num_ref, den_ref, b_ref, g_ref, be_ref),
                         0.0)
        for w_ref, o_ref in ((wl_ref, ol), (wr_ref, orr)):
            o_ref[...] = lax.dot_general(
                hn, w_ref[...], (((1,), (1,)), ((), ())),
                preferred_element_type=jnp.float32,
                precision=lax.Precision.HIGHEST)

    return pl.pallas_call(
        body,
        out_shape=[jax.ShapeDtypeStruct((n, D), jnp.float32)] * 2,
    )(num, den, b, g, be, wl, wr)


def _final(num2, den2, b2, g2, be2, num3, den3, b3, g3, be3):
    n = num2.shape[1]

    def body(n2, d2, b2r, g2r, be2r, n3, d3, b3r, g3r, be3r, out):
        o2 = _combine(n2, d2, b2r, g2r, be2r)
        o3 = _combine(n3, d3, b3r, g3r, be3r)
        out[...] = jnp.maximum(o2 + o3, 0.0)

    return pl.pallas_call(
        body,
        out_shape=jax.ShapeDtypeStruct((n, D), jnp.float32),
    )(num2, den2, b2, g2, be2, num3, den3, b3, g3, be3)


def kernel(x, edge_index, Wl1, Wr1, att1, b1, g1, be1,
           Wl2, Wr2, att2, b2, g2, be2,
           Wl3, Wr3, att3, b3, g3, be3):
    n = x.shape[0]
    e = edge_index.shape[1]
    e_total = e + n
    ch = -(-e_total // (NC * NS * CHUNK))
    e_pad = NC * NS * CHUNK * ch

    idt = edge_index.dtype
    loop = jnp.arange(n, dtype=idt)
    padz = jnp.zeros((e_pad - e_total,), idt)
    src = jnp.concatenate([edge_index[0], loop, padz]).astype(jnp.int32)
    dst = jnp.concatenate([edge_index[1], loop, padz]).astype(jnp.int32)

    npad = -(-n // (NS * 8)) * (NS * 8)
    edge_pass = _make_edge_pass(n, npad, e_total, ch)

    def split(acc):
        num, den = acc
        return num[:, :n], den.reshape(NC, npad)[:, :n].T

    def row(v):
        return v.reshape(1, D)

    xl1, xr1, xl3, xr3 = _mm4(x, Wl1, Wr1, Wl3, Wr3)
    num1, den1 = split(edge_pass(xl1, xr1, src, dst, att1))
    xl2, xr2 = _mid(num1, den1, row(b1), row(g1), row(be1), Wl2, Wr2)
    num2, den2 = split(edge_pass(xl2, xr2, src, dst, att2))
    num3, den3 = split(edge_pass(xl3, xr3, src, dst, att3))
    return _final(num2, den2, row(b2), row(g2), row(be2),
                  num3, den3, row(b3), row(g3), row(be3))


# speedup vs baseline: 2.1900x; 1.1175x over previous
import functools

import jax
import jax.numpy as jnp
from jax import lax
from jax.experimental import pallas as pl
from jax.experimental.pallas import tpu as pltpu
from jax.experimental.pallas import tpu_sc as plsc

NC = 2
NS = 16
LANES = 16
CHUNK = 64
D = 128


def _make_edge_pass(n, npad, e_total, ch):
    assert ch % 2 == 0
    t_per_w = ch * CHUNK
    rows_per_sub = npad // NS
    nd = npad // D
    ng = CHUNK // LANES
    mesh = plsc.VectorSubcoreMesh(
        core_axis_name="c", subcore_axis_name="s",
        num_cores=NC, num_subcores=NS)

    idx_t = pltpu.VMEM((CHUNK,), jnp.int32)
    row_t = pltpu.VMEM((CHUNK, D), jnp.float32)

    @functools.partial(
        pl.kernel,
        out_type=[jax.ShapeDtypeStruct((NC, npad, D), jnp.float32),
                  jax.ShapeDtypeStruct((NC, nd, D), jnp.float32)],
        mesh=mesh,
        compiler_params=pltpu.CompilerParams(needs_layout_passes=False),
        scratch_types=[
            idx_t, idx_t, idx_t, idx_t,
            idx_t, idx_t, idx_t, idx_t,
            row_t, row_t, row_t, row_t,
            pltpu.VMEM((D,), jnp.float32),
            pltpu.VMEM_SHARED((npad, D), jnp.float32),
            pltpu.VMEM_SHARED((nd, D), jnp.float32),
        ] + [pltpu.SemaphoreType.DMA] * 10)
    def edge_kernel(xl_hbm, xr_hbm, src_hbm, dst_hbm, att_hbm,
                    num_hbm, den_hbm,
                    sg0, dg0, sg1, dg1, ds0, dh0, ds1, dh1,
                    rl0, rr0, rl1, rr1, att_v, num_acc, den_acc,
                    si0, si1, gl0, gl1, gr0, gr1, sn0, sn1, sd0, sd1):
        c = lax.axis_index("c")
        s = lax.axis_index("s")
        wid = c * NS + s
        lane = jnp.arange(LANES, dtype=jnp.int32)
        zeros16 = jnp.zeros((LANES,), jnp.float32)
        izeros16 = jnp.zeros((LANES,), jnp.int32)
        eids = [lane + (g * LANES) for g in range(ng)]
        B0 = (sg0, dg0, ds0, dh0, rl0, rr0, si0, gl0, gr0, sn0, sd0)
        B1 = (sg1, dg1, ds1, dh1, rl1, rr1, si1, gl1, gr1, sn1, sd1)

        def issue_idx(j, b):
            sg, dg, _, _, _, _, si = b[:7]
            base = wid * t_per_w + j * CHUNK
            pltpu.async_copy(src_hbm.at[pl.ds(base, CHUNK)], sg, si)
            pltpu.async_copy(dst_hbm.at[pl.ds(base, CHUNK)], dg, si)

        def wait_idx(b):
            sg, dg, _, _, _, _, si = b[:7]
            pltpu.make_async_copy(src_hbm.at[pl.ds(0, CHUNK)], sg,
                                  si).wait()
            pltpu.make_async_copy(dst_hbm.at[pl.ds(0, CHUNK)], dg,
                                  si).wait()

        def issue_gathers(b):
            sg, dg, _, _, rl, rr, _, gl, gr = b[:9]
            pltpu.async_copy(xl_hbm.at[sg], rl, gl)
            pltpu.async_copy(xr_hbm.at[dg], rr, gr)

        def wait_gathers(b):
            sg, dg, _, _, rl, rr, _, gl, gr = b[:9]
            pltpu.make_async_copy(xl_hbm.at[sg], rl, gl).wait()
            pltpu.make_async_copy(xr_hbm.at[dg], rr, gr).wait()

        def issue_scat_den(b):
            rr, _, _, _, _ = b[5], b[6], b[7], b[8], b[9]
            pltpu.async_copy(b[5], den_acc.at[b[3]], b[10], add=True)

        def issue_scat_num(b):
            pltpu.async_copy(b[4], num_acc.at[b[2]], b[9], add=True)

        def wait_scatters(b):
            pltpu.make_async_copy(b[4], num_acc.at[b[2]], b[9]).wait()
            pltpu.make_async_copy(b[5], den_acc.at[b[3]], b[10]).wait()

        def zero_rows(ref):
            def zr(r, carry):
                for i in range(D // LANES):
                    ref[r, pl.ds(i * LANES, LANES)] = zeros16
                return carry
            lax.fori_loop(0, CHUNK, zr, 0)

        def half(j, P, Q):
            wait_scatters(Q)
            wait_idx(Q)
            issue_gathers(Q)
            wait_gathers(P)
            sg, dg, dsS, dhS, rl, rr = P[:6]
            los = []
            for g in range(ng):
                dgv = dg[pl.ds(g * LANES, LANES)]
                dsS[pl.ds(g * LANES, LANES)] = dgv
                dhS[pl.ds(g * LANES, LANES)] = jnp.right_shift(dgv, 7)
                los.append(jnp.bitwise_and(dgv, D - 1))
            issue_idx(j + 2, P)

            def kbody(k, accs):
                kv = jnp.full((LANES,), k, dtype=jnp.int32)
                ak = plsc.load_gather(att_v, [kv])
                out = []
                for g in range(ng):
                    vl = plsc.load_gather(rl, [eids[g], kv])
                    vr = plsc.load_gather(rr, [eids[g], kv])
                    m = vl + vr
                    a = jnp.maximum(m, 0.2 * m)
                    out.append(accs[g] + ak * a)
                return tuple(out)
            accs = lax.fori_loop(0, D, kbody,
                                 tuple([zeros16] * ng), unroll=4)
            base = wid * t_per_w + j * CHUNK
            ps = []
            for g in range(ng):
                gid = base + (g * LANES) + lane
                ps.append(jnp.where(gid < e_total,
                                    jnp.exp(accs[g]), 0.0))

            zero_rows(rr)
            for g in range(ng):
                plsc.store_scatter(rr, [eids[g], los[g]], ps[g])
            issue_scat_den(P)

            def wcol(k, carry):
                kv = jnp.full((LANES,), k, dtype=jnp.int32)
                for g in range(ng):
                    vl = plsc.load_gather(rl, [eids[g], kv])
                    plsc.store_scatter(rl, [eids[g], kv], vl * ps[g])
                return carry
            lax.fori_loop(0, D, wcol, 0, unroll=4)
            issue_scat_num(P)

        pltpu.sync_copy(att_hbm, att_v)
        for ref in (rl0, rr0, rl1, rr1):
            zero_rows(ref)
        row0 = pl.multiple_of(s * rows_per_sub, 8)
        for off in range(0, rows_per_sub, CHUNK):
            size = min(CHUNK, rows_per_sub - off)
            pltpu.sync_copy(rl0.at[pl.ds(0, size)],
                            num_acc.at[pl.ds(row0 + off, size)])
        @pl.when(s == 0)
        def _():
            for off in range(0, nd, CHUNK):
                size = min(CHUNK, nd - off)
                pltpu.sync_copy(rl0.at[pl.ds(0, size)],
                                den_acc.at[pl.ds(off, size)])
        plsc.subcore_barrier()

        for g in range(ng):
            ds1[pl.ds(g * LANES, LANES)] = izeros16
            dh1[pl.ds(g * LANES, LANES)] = izeros16
        issue_scat_num(B1)
        issue_scat_den(B1)
        issue_idx(0, B0)
        issue_idx(1, B1)
        wait_idx(B0)
        issue_gathers(B0)

        def pair(p, carry):
            j = p * 2
            half(j, B0, B1)
            half(j + 1, B1, B0)
            return carry
        lax.fori_loop(0, ch // 2, pair, 0)

        wait_gathers(B0)
        wait_idx(B1)
        wait_scatters(B1)
        plsc.subcore_barrier()
        pltpu.sync_copy(num_acc.at[pl.ds(row0, rows_per_sub)],
                        num_hbm.at[c, pl.ds(row0, rows_per_sub)])
        @pl.when(s == 0)
        def _():
            pltpu.sync_copy(den_acc, den_hbm.at[c])

    return edge_kernel


def _mm4(x, w1, w2, w3, w4):
    n = x.shape[0]
    blk = 1000
    grid = n // blk

    def body(x_ref, w1_ref, w2_ref, w3_ref, w4_ref, o1, o2, o3, o4):
        xb = x_ref[...]
        for w_ref, o_ref in ((w1_ref, o1), (w2_ref, o2),
                             (w3_ref, o3), (w4_ref, o4)):
            o_ref[...] = lax.dot_general(
                xb, w_ref[...], (((1,), (1,)), ((), ())),
                preferred_element_type=jnp.float32,
                precision=lax.Precision.HIGHEST)

    wspec = pl.BlockSpec((D, D), lambda i: (0, 0))
    return pl.pallas_call(
        body,
        grid=(grid,),
        in_specs=[pl.BlockSpec((blk, D), lambda i: (i, 0))] + [wspec] * 4,
        out_specs=[pl.BlockSpec((blk, D), lambda i: (i, 0))] * 4,
        out_shape=[jax.ShapeDtypeStruct((n, D), jnp.float32)] * 4,
    )(x, w1, w2, w3, w4)


def _combine(num_ref, den_ref, b_ref, g_ref, be_ref):
    a = num_ref[0]
    den = den_ref[:, 0:1]
    for i in range(1, NC):
        a = a + num_ref[i]
        den = den + den_ref[:, i:i + 1]
    h = a / (den + 1e-16) + b_ref[...]
    mean = jnp.mean(h, axis=0, keepdims=True)
    var = jnp.mean((h - mean) ** 2, axis=0, keepdims=True)
    return (h - mean) * lax.rsqrt(var + 1e-5) * g_ref[...] + be_ref[...]


def _mid(num, den, b, g, be, wl, wr):
    n = num.shape[1]

    def body(num_ref, den_ref, b_ref, g_ref, be_ref, wl_ref, wr_ref,
             ol, orr):
        hn = jnp.maximum(_combine(num_ref, den_ref, b_ref, g_ref, be_ref),
                         0.0)
        for w_ref, o_ref in ((wl_ref, ol), (wr_ref, orr)):
            o_ref[...] = lax.dot_general(
                hn, w_ref[...], (((1,), (1,)), ((), ())),
                preferred_element_type=jnp.float32,
                precision=lax.Precision.HIGHEST)

    return pl.pallas_call(
        body,
        out_shape=[jax.ShapeDtypeStruct((n, D), jnp.float32)] * 2,
    )(num, den, b, g, be, wl, wr)


def _final(num2, den2, b2, g2, be2, num3, den3, b3, g3, be3):
    n = num2.shape[1]

    def body(n2, d2, b2r, g2r, be2r, n3, d3, b3r, g3r, be3r, out):
        o2 = _combine(n2, d2, b2r, g2r, be2r)
        o3 = _combine(n3, d3, b3r, g3r, be3r)
        out[...] = jnp.maximum(o2 + o3, 0.0)

    return pl.pallas_call(
        body,
        out_shape=jax.ShapeDtypeStruct((n, D), jnp.float32),
    )(num2, den2, b2, g2, be2, num3, den3, b3, g3, be3)


def kernel(x, edge_index, Wl1, Wr1, att1, b1, g1, be1,
           Wl2, Wr2, att2, b2, g2, be2,
           Wl3, Wr3, att3, b3, g3, be3):
    n = x.shape[0]
    e = edge_index.shape[1]
    e_total = e + n
    ch = -(-e_total // (NC * NS * CHUNK))
    ch = ch + (ch % 2)
    e_pad = NC * NS * CHUNK * ch + 2 * CHUNK

    idt = edge_index.dtype
    loop = jnp.arange(n, dtype=idt)
    padz = jnp.zeros((e_pad - e_total,), idt)
    src = jnp.concatenate([edge_index[0], loop, padz]).astype(jnp.int32)
    dst = jnp.concatenate([edge_index[1], loop, padz]).astype(jnp.int32)

    npad = -(-n // (NS * 8)) * (NS * 8)
    edge_pass = _make_edge_pass(n, npad, e_total, ch)

    def split(acc):
        num, den = acc
        return num[:, :n], den.reshape(NC, npad)[:, :n].T

    def row(v):
        return v.reshape(1, D)

    xl1, xr1, xl3, xr3 = _mm4(x, Wl1, Wr1, Wl3, Wr3)
    num1, den1 = split(edge_pass(xl1, xr1, src, dst, att1))
    xl2, xr2 = _mid(num1, den1, row(b1), row(g1), row(be1), Wl2, Wr2)
    num2, den2 = split(edge_pass(xl2, xr2, src, dst, att2))
    num3, den3 = split(edge_pass(xl3, xr3, src, dst, att3))
    return _final(num2, den2, row(b2), row(g2), row(be2),
                  num3, den3, row(b3), row(g3), row(be3))


# speedup vs baseline: 11.2564x; 5.1399x over previous
import functools

import jax
import jax.numpy as jnp
from jax import lax
from jax.experimental import pallas as pl
from jax.experimental.pallas import tpu as pltpu
from jax.experimental.pallas import tpu_sc as plsc

NC = 2
NS = 16
LANES = 16
CHUNK = 64
D = 128


def _make_edge_pass(n, npad, e_total, ch):
    assert ch % 2 == 0
    t_per_w = ch * CHUNK
    rows_per_sub = npad // NS
    nd = npad // D
    ng = CHUNK // LANES
    mesh = plsc.VectorSubcoreMesh(
        core_axis_name="c", subcore_axis_name="s",
        num_cores=NC, num_subcores=NS)

    idx_t = pltpu.VMEM((CHUNK,), jnp.int32)
    row_t = pltpu.VMEM((CHUNK, D), jnp.float32)

    @functools.partial(
        pl.kernel,
        out_type=[jax.ShapeDtypeStruct((NC, npad, D), jnp.float32),
                  jax.ShapeDtypeStruct((NC, nd, D), jnp.float32)],
        mesh=mesh,
        compiler_params=pltpu.CompilerParams(needs_layout_passes=False),
        scratch_types=[
            idx_t, idx_t, idx_t, idx_t,
            idx_t, idx_t, idx_t, idx_t,
            row_t, row_t, row_t, row_t,
            pltpu.VMEM((D,), jnp.float32),
            pltpu.VMEM_SHARED((npad, D), jnp.float32),
            pltpu.VMEM_SHARED((nd, D), jnp.float32),
        ] + [pltpu.SemaphoreType.DMA] * 10)
    def edge_kernel(xl_hbm, xr_hbm, src_hbm, dst_hbm, att_hbm,
                    num_hbm, den_hbm,
                    sg0, dg0, sg1, dg1, ds0, dh0, ds1, dh1,
                    rl0, rr0, rl1, rr1, att_v, num_acc, den_acc,
                    si0, si1, gl0, gl1, gr0, gr1, sn0, sn1, sd0, sd1):
        c = lax.axis_index("c")
        s = lax.axis_index("s")
        wid = c * NS + s
        lane = jnp.arange(LANES, dtype=jnp.int32)
        zeros16 = jnp.zeros((LANES,), jnp.float32)
        izeros16 = jnp.zeros((LANES,), jnp.int32)
        eids = [lane + (g * LANES) for g in range(ng)]
        B0 = (sg0, dg0, ds0, dh0, rl0, rr0, si0, gl0, gr0, sn0, sd0)
        B1 = (sg1, dg1, ds1, dh1, rl1, rr1, si1, gl1, gr1, sn1, sd1)

        def issue_idx(j, b):
            sg, dg, _, _, _, _, si = b[:7]
            base = wid * t_per_w + j * CHUNK
            pltpu.async_copy(src_hbm.at[pl.ds(base, CHUNK)], sg, si)
            pltpu.async_copy(dst_hbm.at[pl.ds(base, CHUNK)], dg, si)

        def wait_idx(b):
            sg, dg, _, _, _, _, si = b[:7]
            pltpu.make_async_copy(src_hbm.at[pl.ds(0, CHUNK)], sg,
                                  si).wait()
            pltpu.make_async_copy(dst_hbm.at[pl.ds(0, CHUNK)], dg,
                                  si).wait()

        def issue_gathers(b):
            sg, dg, _, _, rl, rr, _, gl, gr = b[:9]
            pltpu.async_copy(xl_hbm.at[sg], rl, gl)
            pltpu.async_copy(xr_hbm.at[dg], rr, gr)

        def wait_gathers(b):
            sg, dg, _, _, rl, rr, _, gl, gr = b[:9]
            pltpu.make_async_copy(xl_hbm.at[sg], rl, gl).wait()
            pltpu.make_async_copy(xr_hbm.at[dg], rr, gr).wait()

        def issue_scat_den(b):
            rr, _, _, _, _ = b[5], b[6], b[7], b[8], b[9]
            pltpu.async_copy(b[5], den_acc.at[b[3]], b[10], add=True)

        def issue_scat_num(b):
            pltpu.async_copy(b[4], num_acc.at[b[2]], b[9], add=True)

        def wait_scatters(b):
            pltpu.make_async_copy(b[4], num_acc.at[b[2]], b[9]).wait()
            pltpu.make_async_copy(b[5], den_acc.at[b[3]], b[10]).wait()

        def zero_rows(ref):
            def zr(r, carry):
                for i in range(D // LANES):
                    ref[r, pl.ds(i * LANES, LANES)] = zeros16
                return carry
            lax.fori_loop(0, CHUNK, zr, 0)

        def half(j, P, Q):
            wait_scatters(Q)
            wait_idx(Q)
            issue_gathers(Q)
            wait_gathers(P)
            sg, dg, dsS, dhS, rl, rr = P[:6]
            los = []
            for g in range(ng):
                dgv = dg[pl.ds(g * LANES, LANES)]
                dsS[pl.ds(g * LANES, LANES)] = dgv
                dhS[pl.ds(g * LANES, LANES)] = jnp.right_shift(dgv, 7)
                los.append(jnp.bitwise_and(dgv, D - 1))
            issue_idx(j + 2, P)

            base = wid * t_per_w + j * CHUNK

            def ebody(e, pacc):
                acc = zeros16
                vls = []
                for kk in range(D // LANES):
                    vl = rl[e, pl.ds(kk * LANES, LANES)]
                    vr = rr[e, pl.ds(kk * LANES, LANES)]
                    m = vl + vr
                    a = jnp.maximum(m, 0.2 * m)
                    acc = acc + att8[kk] * a
                    vls.append(vl)
                cs = plsc.cumsum(acc)
                pe = jnp.exp(jnp.full((LANES,), cs[15]))
                pe = jnp.where(base + e < e_total, pe, zeros16)
                for kk in range(D // LANES):
                    rl[e, pl.ds(kk * LANES, LANES)] = vls[kk] * pe
                el = jnp.bitwise_and(e, LANES - 1)
                eg = jnp.right_shift(e, 4)
                lm = lane == el
                return tuple(
                    jnp.where(jnp.logical_and(lm, eg == g), pe, pacc[g])
                    for g in range(ng))
            paccs = lax.fori_loop(0, CHUNK, ebody,
                                  tuple([zeros16] * ng), unroll=2)

            zero_rows(rr)
            for g in range(ng):
                plsc.store_scatter(rr, [eids[g], los[g]], paccs[g])
            issue_scat_den(P)
            issue_scat_num(P)

        pltpu.sync_copy(att_hbm, att_v)
        att8 = [att_v[pl.ds(kk * LANES, LANES)]
                for kk in range(D // LANES)]
        for ref in (rl0, rr0, rl1, rr1):
            zero_rows(ref)
        row0 = pl.multiple_of(s * rows_per_sub, 8)
        for off in range(0, rows_per_sub, CHUNK):
            size = min(CHUNK, rows_per_sub - off)
            pltpu.sync_copy(rl0.at[pl.ds(0, size)],
                            num_acc.at[pl.ds(row0 + off, size)])
        @pl.when(s == 0)
        def _():
            for off in range(0, nd, CHUNK):
                size = min(CHUNK, nd - off)
                pltpu.sync_copy(rl0.at[pl.ds(0, size)],
                                den_acc.at[pl.ds(off, size)])
        plsc.subcore_barrier()

        for g in range(ng):
            ds1[pl.ds(g * LANES, LANES)] = izeros16
            dh1[pl.ds(g * LANES, LANES)] = izeros16
        issue_scat_num(B1)
        issue_scat_den(B1)
        issue_idx(0, B0)
        issue_idx(1, B1)
        wait_idx(B0)
        issue_gathers(B0)

        def pair(p, carry):
            j = p * 2
            half(j, B0, B1)
            half(j + 1, B1, B0)
            return carry
        lax.fori_loop(0, ch // 2, pair, 0)

        wait_gathers(B0)
        wait_idx(B1)
        wait_scatters(B1)
        plsc.subcore_barrier()
        pltpu.sync_copy(num_acc.at[pl.ds(row0, rows_per_sub)],
                        num_hbm.at[c, pl.ds(row0, rows_per_sub)])
        @pl.when(s == 0)
        def _():
            pltpu.sync_copy(den_acc, den_hbm.at[c])

    return edge_kernel


def _mm4(x, w1, w2, w3, w4):
    n = x.shape[0]
    blk = 1000
    grid = n // blk

    def body(x_ref, w1_ref, w2_ref, w3_ref, w4_ref, o1, o2, o3, o4):
        xb = x_ref[...]
        for w_ref, o_ref in ((w1_ref, o1), (w2_ref, o2),
                             (w3_ref, o3), (w4_ref, o4)):
            o_ref[...] = lax.dot_general(
                xb, w_ref[...], (((1,), (1,)), ((), ())),
                preferred_element_type=jnp.float32,
                precision=lax.Precision.HIGHEST)

    wspec = pl.BlockSpec((D, D), lambda i: (0, 0))
    return pl.pallas_call(
        body,
        grid=(grid,),
        in_specs=[pl.BlockSpec((blk, D), lambda i: (i, 0))] + [wspec] * 4,
        out_specs=[pl.BlockSpec((blk, D), lambda i: (i, 0))] * 4,
        out_shape=[jax.ShapeDtypeStruct((n, D), jnp.float32)] * 4,
    )(x, w1, w2, w3, w4)


def _combine(num_ref, den_ref, b_ref, g_ref, be_ref):
    a = num_ref[0]
    den = den_ref[:, 0:1]
    for i in range(1, NC):
        a = a + num_ref[i]
        den = den + den_ref[:, i:i + 1]
    h = a / (den + 1e-16) + b_ref[...]
    mean = jnp.mean(h, axis=0, keepdims=True)
    var = jnp.mean((h - mean) ** 2, axis=0, keepdims=True)
    return (h - mean) * lax.rsqrt(var + 1e-5) * g_ref[...] + be_ref[...]


def _mid(num, den, b, g, be, wl, wr):
    n = num.shape[1]

    def body(num_ref, den_ref, b_ref, g_ref, be_ref, wl_ref, wr_ref,
             ol, orr):
        hn = jnp.maximum(_combine(num_ref, den_ref, b_ref, g_ref, be_ref),
                         0.0)
        for w_ref, o_ref in ((wl_ref, ol), (wr_ref, orr)):
            o_ref[...] = lax.dot_general(
                hn, w_ref[...], (((1,), (1,)), ((), ())),
                preferred_element_type=jnp.float32,
                precision=lax.Precision.HIGHEST)

    return pl.pallas_call(
        body,
        out_shape=[jax.ShapeDtypeStruct((n, D), jnp.float32)] * 2,
    )(num, den, b, g, be, wl, wr)


def _final(num2, den2, b2, g2, be2, num3, den3, b3, g3, be3):
    n = num2.shape[1]

    def body(n2, d2, b2r, g2r, be2r, n3, d3, b3r, g3r, be3r, out):
        o2 = _combine(n2, d2, b2r, g2r, be2r)
        o3 = _combine(n3, d3, b3r, g3r, be3r)
        out[...] = jnp.maximum(o2 + o3, 0.0)

    return pl.pallas_call(
        body,
        out_shape=jax.ShapeDtypeStruct((n, D), jnp.float32),
    )(num2, den2, b2, g2, be2, num3, den3, b3, g3, be3)


def kernel(x, edge_index, Wl1, Wr1, att1, b1, g1, be1,
           Wl2, Wr2, att2, b2, g2, be2,
           Wl3, Wr3, att3, b3, g3, be3):
    n = x.shape[0]
    e = edge_index.shape[1]
    e_total = e + n
    ch = -(-e_total // (NC * NS * CHUNK))
    ch = ch + (ch % 2)
    e_pad = NC * NS * CHUNK * ch + 2 * CHUNK

    idt = edge_index.dtype
    loop = jnp.arange(n, dtype=idt)
    padz = jnp.zeros((e_pad - e_total,), idt)
    src = jnp.concatenate([edge_index[0], loop, padz]).astype(jnp.int32)
    dst = jnp.concatenate([edge_index[1], loop, padz]).astype(jnp.int32)

    npad = -(-n // (NS * 8)) * (NS * 8)
    edge_pass = _make_edge_pass(n, npad, e_total, ch)

    def split(acc):
        num, den = acc
        return num[:, :n], den.reshape(NC, npad)[:, :n].T

    def row(v):
        return v.reshape(1, D)

    xl1, xr1, xl3, xr3 = _mm4(x, Wl1, Wr1, Wl3, Wr3)
    num1, den1 = split(edge_pass(xl1, xr1, src, dst, att1))
    xl2, xr2 = _mid(num1, den1, row(b1), row(g1), row(be1), Wl2, Wr2)
    num2, den2 = split(edge_pass(xl2, xr2, src, dst, att2))
    num3, den3 = split(edge_pass(xl3, xr3, src, dst, att3))
    return _final(num2, den2, row(b2), row(g2), row(be2),
                  num3, den3, row(b3), row(g3), row(be3))


# speedup vs baseline: 11.3239x; 1.0060x over previous
import functools

import jax
import jax.numpy as jnp
from jax import lax
from jax.experimental import pallas as pl
from jax.experimental.pallas import tpu as pltpu
from jax.experimental.pallas import tpu_sc as plsc

NC = 2
NS = 16
LANES = 16
CHUNK = 64
D = 128


def _make_edge_pass(n, npad, e_total, ch):
    assert ch % 2 == 0
    t_per_w = ch * CHUNK
    rows_per_sub = npad // NS
    nd = npad // D
    ng = CHUNK // LANES
    mesh = plsc.VectorSubcoreMesh(
        core_axis_name="c", subcore_axis_name="s",
        num_cores=NC, num_subcores=NS)

    idx_t = pltpu.VMEM((CHUNK,), jnp.int32)
    row_t = pltpu.VMEM((CHUNK, D), jnp.float32)

    @functools.partial(
        pl.kernel,
        out_type=[jax.ShapeDtypeStruct((NC, npad, D), jnp.float32),
                  jax.ShapeDtypeStruct((NC, nd, D), jnp.float32)],
        mesh=mesh,
        compiler_params=pltpu.CompilerParams(needs_layout_passes=False),
        scratch_types=[
            idx_t, idx_t, idx_t, idx_t,
            idx_t, idx_t, idx_t, idx_t,
            row_t, row_t, row_t, row_t,
            pltpu.VMEM((D,), jnp.float32),
            pltpu.VMEM_SHARED((npad, D), jnp.float32),
            pltpu.VMEM_SHARED((nd, D), jnp.float32),
        ] + [pltpu.SemaphoreType.DMA] * 10)
    def edge_kernel(xl_hbm, xr_hbm, src_hbm, dst_hbm, att_hbm,
                    num_hbm, den_hbm,
                    sg0, dg0, sg1, dg1, ds0, dh0, ds1, dh1,
                    rl0, rr0, rl1, rr1, att_v, num_acc, den_acc,
                    si0, si1, gl0, gl1, gr0, gr1, sn0, sn1, sd0, sd1):
        c = lax.axis_index("c")
        s = lax.axis_index("s")
        wid = c * NS + s
        lane = jnp.arange(LANES, dtype=jnp.int32)
        zeros16 = jnp.zeros((LANES,), jnp.float32)
        izeros16 = jnp.zeros((LANES,), jnp.int32)
        eids = [lane + (g * LANES) for g in range(ng)]
        B0 = (sg0, dg0, ds0, dh0, rl0, rr0, si0, gl0, gr0, sn0, sd0)
        B1 = (sg1, dg1, ds1, dh1, rl1, rr1, si1, gl1, gr1, sn1, sd1)

        def issue_idx(j, b):
            sg, dg, _, _, _, _, si = b[:7]
            base = wid * t_per_w + j * CHUNK
            pltpu.async_copy(src_hbm.at[pl.ds(base, CHUNK)], sg, si)
            pltpu.async_copy(dst_hbm.at[pl.ds(base, CHUNK)], dg, si)

        def wait_idx(b):
            sg, dg, _, _, _, _, si = b[:7]
            pltpu.make_async_copy(src_hbm.at[pl.ds(0, CHUNK)], sg,
                                  si).wait()
            pltpu.make_async_copy(dst_hbm.at[pl.ds(0, CHUNK)], dg,
                                  si).wait()

        def issue_gathers(b):
            sg, dg, _, _, rl, rr, _, gl, gr = b[:9]
            pltpu.async_copy(xl_hbm.at[sg], rl, gl)
            pltpu.async_copy(xr_hbm.at[dg], rr, gr)

        def wait_gathers(b):
            sg, dg, _, _, rl, rr, _, gl, gr = b[:9]
            pltpu.make_async_copy(xl_hbm.at[sg], rl, gl).wait()
            pltpu.make_async_copy(xr_hbm.at[dg], rr, gr).wait()

        def issue_scat_den(b):
            rr, _, _, _, _ = b[5], b[6], b[7], b[8], b[9]
            pltpu.async_copy(b[5], den_acc.at[b[3]], b[10], add=True)

        def issue_scat_num(b):
            pltpu.async_copy(b[4], num_acc.at[b[2]], b[9], add=True)

        def wait_scatters(b):
            pltpu.make_async_copy(b[4], num_acc.at[b[2]], b[9]).wait()
            pltpu.make_async_copy(b[5], den_acc.at[b[3]], b[10]).wait()

        def zero_rows(ref):
            def zr(r, carry):
                for i in range(D // LANES):
                    ref[r, pl.ds(i * LANES, LANES)] = zeros16
                return carry
            lax.fori_loop(0, CHUNK, zr, 0)

        def half(j, P, Q):
            wait_scatters(Q)
            wait_idx(Q)
            issue_gathers(Q)
            wait_gathers(P)
            sg, dg, dsS, dhS, rl, rr = P[:6]
            los = []
            for g in range(ng):
                dgv = dg[pl.ds(g * LANES, LANES)]
                dsS[pl.ds(g * LANES, LANES)] = dgv
                dhS[pl.ds(g * LANES, LANES)] = jnp.right_shift(dgv, 7)
                los.append(jnp.bitwise_and(dgv, D - 1))
            issue_idx(j + 2, P)

            base = wid * t_per_w + j * CHUNK

            def ebody(e, pacc):
                acc = zeros16
                vls = []
                for kk in range(D // LANES):
                    vl = rl[e, pl.ds(kk * LANES, LANES)]
                    vr = rr[e, pl.ds(kk * LANES, LANES)]
                    m = vl + vr
                    a = jnp.maximum(m, 0.2 * m)
                    acc = acc + att8[kk] * a
                    vls.append(vl)
                cs = plsc.cumsum(acc)
                pe = jnp.exp(jnp.full((LANES,), cs[15]))
                pe = jnp.where(base + e < e_total, pe, zeros16)
                for kk in range(D // LANES):
                    rl[e, pl.ds(kk * LANES, LANES)] = vls[kk] * pe
                el = jnp.bitwise_and(e, LANES - 1)
                eg = jnp.right_shift(e, 4)
                lm = lane == el
                return tuple(
                    jnp.where(jnp.logical_and(lm, eg == g), pe, pacc[g])
                    for g in range(ng))
            paccs = lax.fori_loop(0, CHUNK, ebody,
                                  tuple([zeros16] * ng), unroll=4)

            zero_rows(rr)
            for g in range(ng):
                plsc.store_scatter(rr, [eids[g], los[g]], paccs[g])
            issue_scat_den(P)
            issue_scat_num(P)

        pltpu.sync_copy(att_hbm, att_v)
        att8 = [att_v[pl.ds(kk * LANES, LANES)]
                for kk in range(D // LANES)]
        for ref in (rl0, rr0, rl1, rr1):
            zero_rows(ref)
        row0 = pl.multiple_of(s * rows_per_sub, 8)
        for off in range(0, rows_per_sub, CHUNK):
            size = min(CHUNK, rows_per_sub - off)
            pltpu.sync_copy(rl0.at[pl.ds(0, size)],
                            num_acc.at[pl.ds(row0 + off, size)])
        @pl.when(s == 0)
        def _():
            for off in range(0, nd, CHUNK):
                size = min(CHUNK, nd - off)
                pltpu.sync_copy(rl0.at[pl.ds(0, size)],
                                den_acc.at[pl.ds(off, size)])
        plsc.subcore_barrier()

        for g in range(ng):
            ds1[pl.ds(g * LANES, LANES)] = izeros16
            dh1[pl.ds(g * LANES, LANES)] = izeros16
        issue_scat_num(B1)
        issue_scat_den(B1)
        issue_idx(0, B0)
        issue_idx(1, B1)
        wait_idx(B0)
        issue_gathers(B0)

        def pair(p, carry):
            j = p * 2
            half(j, B0, B1)
            half(j + 1, B1, B0)
            return carry
        lax.fori_loop(0, ch // 2, pair, 0)

        wait_gathers(B0)
        wait_idx(B1)
        wait_scatters(B1)
        plsc.subcore_barrier()
        pltpu.sync_copy(num_acc.at[pl.ds(row0, rows_per_sub)],
                        num_hbm.at[c, pl.ds(row0, rows_per_sub)])
        @pl.when(s == 0)
        def _():
            pltpu.sync_copy(den_acc, den_hbm.at[c])

    return edge_kernel


def _mm4(x, w1, w2, w3, w4):
    n = x.shape[0]
    blk = 1000
    grid = n // blk

    def body(x_ref, w1_ref, w2_ref, w3_ref, w4_ref, o1, o2, o3, o4):
        xb = x_ref[...]
        for w_ref, o_ref in ((w1_ref, o1), (w2_ref, o2),
                             (w3_ref, o3), (w4_ref, o4)):
            o_ref[...] = lax.dot_general(
                xb, w_ref[...], (((1,), (1,)), ((), ())),
                preferred_element_type=jnp.float32,
                precision=lax.Precision.HIGHEST)

    wspec = pl.BlockSpec((D, D), lambda i: (0, 0))
    return pl.pallas_call(
        body,
        grid=(grid,),
        in_specs=[pl.BlockSpec((blk, D), lambda i: (i, 0))] + [wspec] * 4,
        out_specs=[pl.BlockSpec((blk, D), lambda i: (i, 0))] * 4,
        out_shape=[jax.ShapeDtypeStruct((n, D), jnp.float32)] * 4,
    )(x, w1, w2, w3, w4)


def _combine(num_ref, den_ref, b_ref, g_ref, be_ref):
    a = num_ref[0]
    den = den_ref[:, 0:1]
    for i in range(1, NC):
        a = a + num_ref[i]
        den = den + den_ref[:, i:i + 1]
    h = a / (den + 1e-16) + b_ref[...]
    mean = jnp.mean(h, axis=0, keepdims=True)
    var = jnp.mean((h - mean) ** 2, axis=0, keepdims=True)
    return (h - mean) * lax.rsqrt(var + 1e-5) * g_ref[...] + be_ref[...]


def _mid(num, den, b, g, be, wl, wr):
    n = num.shape[1]

    def body(num_ref, den_ref, b_ref, g_ref, be_ref, wl_ref, wr_ref,
             ol, orr):
        hn = jnp.maximum(_combine(num_ref, den_ref, b_ref, g_ref, be_ref),
                         0.0)
        for w_ref, o_ref in ((wl_ref, ol), (wr_ref, orr)):
            o_ref[...] = lax.dot_general(
                hn, w_ref[...], (((1,), (1,)), ((), ())),
                preferred_element_type=jnp.float32,
                precision=lax.Precision.HIGHEST)

    return pl.pallas_call(
        body,
        out_shape=[jax.ShapeDtypeStruct((n, D), jnp.float32)] * 2,
    )(num, den, b, g, be, wl, wr)


def _final(num2, den2, b2, g2, be2, num3, den3, b3, g3, be3):
    n = num2.shape[1]

    def body(n2, d2, b2r, g2r, be2r, n3, d3, b3r, g3r, be3r, out):
        o2 = _combine(n2, d2, b2r, g2r, be2r)
        o3 = _combine(n3, d3, b3r, g3r, be3r)
        out[...] = jnp.maximum(o2 + o3, 0.0)

    return pl.pallas_call(
        body,
        out_shape=jax.ShapeDtypeStruct((n, D), jnp.float32),
    )(num2, den2, b2, g2, be2, num3, den3, b3, g3, be3)


def kernel(x, edge_index, Wl1, Wr1, att1, b1, g1, be1,
           Wl2, Wr2, att2, b2, g2, be2,
           Wl3, Wr3, att3, b3, g3, be3):
    n = x.shape[0]
    e = edge_index.shape[1]
    e_total = e + n
    ch = -(-e_total // (NC * NS * CHUNK))
    ch = ch + (ch % 2)
    e_pad = NC * NS * CHUNK * ch + 2 * CHUNK

    idt = edge_index.dtype
    loop = jnp.arange(n, dtype=idt)
    padz = jnp.zeros((e_pad - e_total,), idt)
    src = jnp.concatenate([edge_index[0], loop, padz]).astype(jnp.int32)
    dst = jnp.concatenate([edge_index[1], loop, padz]).astype(jnp.int32)

    npad = -(-n // (NS * 8)) * (NS * 8)
    edge_pass = _make_edge_pass(n, npad, e_total, ch)

    def split(acc):
        num, den = acc
        return num[:, :n], den.reshape(NC, npad)[:, :n].T

    def row(v):
        return v.reshape(1, D)

    xl1, xr1, xl3, xr3 = _mm4(x, Wl1, Wr1, Wl3, Wr3)
    num1, den1 = split(edge_pass(xl1, xr1, src, dst, att1))
    xl2, xr2 = _mid(num1, den1, row(b1), row(g1), row(be1), Wl2, Wr2)
    num2, den2 = split(edge_pass(xl2, xr2, src, dst, att2))
    num3, den3 = split(edge_pass(xl3, xr3, src, dst, att3))
    return _final(num2, den2, row(b2), row(g2), row(be2),
                  num3, den3, row(b3), row(g3), row(be3))


# speedup vs baseline: 12.6601x; 1.1180x over previous
import functools

import jax
import jax.numpy as jnp
from jax import lax
from jax.experimental import pallas as pl
from jax.experimental.pallas import tpu as pltpu
from jax.experimental.pallas import tpu_sc as plsc

NC = 2
NS = 16
LANES = 16
CHUNK = 64
D = 128


def _make_edge_pass(n, npad, e_total, ch):
    assert ch % 2 == 0
    t_per_w = ch * CHUNK
    rows_per_sub = npad // NS
    nd = npad // LANES
    ng = CHUNK // LANES
    mesh = plsc.VectorSubcoreMesh(
        core_axis_name="c", subcore_axis_name="s",
        num_cores=NC, num_subcores=NS)

    idx_t = pltpu.VMEM((CHUNK,), jnp.int32)
    row_t = pltpu.VMEM((CHUNK, D), jnp.float32)

    @functools.partial(
        pl.kernel,
        out_type=[jax.ShapeDtypeStruct((NC, npad, D), jnp.float32),
                  jax.ShapeDtypeStruct((NC, nd, LANES), jnp.float32)],
        mesh=mesh,
        compiler_params=pltpu.CompilerParams(needs_layout_passes=False),
        scratch_types=[
            idx_t, idx_t, idx_t, idx_t,
            idx_t, idx_t, idx_t, idx_t,
            pltpu.VMEM((CHUNK, LANES), jnp.float32),
            row_t, row_t, row_t, row_t,
            pltpu.VMEM((D,), jnp.float32),
            pltpu.VMEM_SHARED((npad, D), jnp.float32),
            pltpu.VMEM_SHARED((nd, LANES), jnp.float32),
        ] + [pltpu.SemaphoreType.DMA] * 10)
    def edge_kernel(xl_hbm, xr_hbm, src_hbm, dst_hbm, att_hbm,
                    num_hbm, den_hbm,
                    sg0, dg0, sg1, dg1, ds0, dh0, ds1, dh1, db,
                    rl0, rr0, rl1, rr1, att_v, num_acc, den_acc,
                    si0, si1, gl0, gl1, gr0, gr1, sn0, sn1, sd0, sd1):
        c = lax.axis_index("c")
        s = lax.axis_index("s")
        wid = c * NS + s
        lane = jnp.arange(LANES, dtype=jnp.int32)
        zeros16 = jnp.zeros((LANES,), jnp.float32)
        izeros16 = jnp.zeros((LANES,), jnp.int32)
        eids = [lane + (g * LANES) for g in range(ng)]
        B0 = (sg0, dg0, ds0, dh0, rl0, rr0, si0, gl0, gr0, sn0, sd0)
        B1 = (sg1, dg1, ds1, dh1, rl1, rr1, si1, gl1, gr1, sn1, sd1)

        def issue_idx(j, b):
            sg, dg, _, _, _, _, si = b[:7]
            base = wid * t_per_w + j * CHUNK
            pltpu.async_copy(src_hbm.at[pl.ds(base, CHUNK)], sg, si)
            pltpu.async_copy(dst_hbm.at[pl.ds(base, CHUNK)], dg, si)

        def wait_idx(b):
            sg, dg, _, _, _, _, si = b[:7]
            pltpu.make_async_copy(src_hbm.at[pl.ds(0, CHUNK)], sg,
                                  si).wait()
            pltpu.make_async_copy(dst_hbm.at[pl.ds(0, CHUNK)], dg,
                                  si).wait()

        def issue_gathers(b):
            sg, dg, _, _, rl, rr, _, gl, gr = b[:9]
            pltpu.async_copy(xl_hbm.at[sg], rl, gl)
            pltpu.async_copy(xr_hbm.at[dg], rr, gr)

        def wait_gathers(b):
            sg, dg, _, _, rl, rr, _, gl, gr = b[:9]
            pltpu.make_async_copy(xl_hbm.at[sg], rl, gl).wait()
            pltpu.make_async_copy(xr_hbm.at[dg], rr, gr).wait()

        def issue_scat_den(b):
            pltpu.async_copy(db, den_acc.at[b[3]], b[10], add=True)

        def wait_den(b):
            pltpu.make_async_copy(db, den_acc.at[b[3]], b[10]).wait()
            for g in range(ng):
                dgv = b[2][pl.ds(g * LANES, LANES)]
                plsc.store_scatter(db, [eids[g],
                                        jnp.bitwise_and(dgv, LANES - 1)],
                                   zeros16)

        def issue_scat_num(b):
            pltpu.async_copy(b[4], num_acc.at[b[2]], b[9], add=True)

        def wait_scatters(b):
            pltpu.make_async_copy(b[4], num_acc.at[b[2]], b[9]).wait()

        def zero_rows(ref):
            def zr(r, carry):
                for i in range(D // LANES):
                    ref[r, pl.ds(i * LANES, LANES)] = zeros16
                return carry
            lax.fori_loop(0, CHUNK, zr, 0)

        def half(j, P, Q):
            wait_scatters(Q)
            wait_idx(Q)
            issue_gathers(Q)
            wait_gathers(P)
            sg, dg, dsS, dhS, rl, rr = P[:6]
            los = []
            for g in range(ng):
                dgv = dg[pl.ds(g * LANES, LANES)]
                dsS[pl.ds(g * LANES, LANES)] = dgv
                dhS[pl.ds(g * LANES, LANES)] = jnp.right_shift(dgv, 4)
                los.append(jnp.bitwise_and(dgv, LANES - 1))
            issue_idx(j + 2, P)

            base = wid * t_per_w + j * CHUNK

            def ebody(e, pacc):
                acc = zeros16
                vls = []
                for kk in range(D // LANES):
                    vl = rl[e, pl.ds(kk * LANES, LANES)]
                    vr = rr[e, pl.ds(kk * LANES, LANES)]
                    m = vl + vr
                    a = jnp.maximum(m, 0.2 * m)
                    acc = acc + att8[kk] * a
                    vls.append(vl)
                cs = plsc.cumsum(acc)
                pe = jnp.exp(jnp.full((LANES,), cs[15]))
                pe = jnp.where(base + e < e_total, pe, zeros16)
                for kk in range(D // LANES):
                    rl[e, pl.ds(kk * LANES, LANES)] = vls[kk] * pe
                el = jnp.bitwise_and(e, LANES - 1)
                eg = jnp.right_shift(e, 4)
                lm = lane == el
                return tuple(
                    jnp.where(jnp.logical_and(lm, eg == g), pe, pacc[g])
                    for g in range(ng))
            paccs = lax.fori_loop(0, CHUNK, ebody,
                                  tuple([zeros16] * ng), unroll=4)

            wait_den(Q)
            for g in range(ng):
                plsc.store_scatter(db, [eids[g], los[g]], paccs[g])
            issue_scat_den(P)
            issue_scat_num(P)

        pltpu.sync_copy(att_hbm, att_v)
        att8 = [att_v[pl.ds(kk * LANES, LANES)]
                for kk in range(D // LANES)]
        for ref in (rl0, rr0, rl1, rr1):
            zero_rows(ref)
        def zd(r, carry):
            db[r, pl.ds(0, LANES)] = zeros16
            return carry
        lax.fori_loop(0, CHUNK, zd, 0)
        row0 = pl.multiple_of(s * rows_per_sub, 8)
        for off in range(0, rows_per_sub, CHUNK):
            size = min(CHUNK, rows_per_sub - off)
            pltpu.sync_copy(rl0.at[pl.ds(0, size)],
                            num_acc.at[pl.ds(row0 + off, size)])
        @pl.when(s == 0)
        def _():
            for off in range(0, nd, CHUNK):
                size = min(CHUNK, nd - off)
                pltpu.sync_copy(db.at[pl.ds(0, size)],
                                den_acc.at[pl.ds(off, size)])
        plsc.subcore_barrier()

        for g in range(ng):
            ds1[pl.ds(g * LANES, LANES)] = izeros16
            dh1[pl.ds(g * LANES, LANES)] = izeros16
        issue_scat_num(B1)
        issue_scat_den(B1)
        issue_idx(0, B0)
        issue_idx(1, B1)
        wait_idx(B0)
        issue_gathers(B0)

        def pair(p, carry):
            j = p * 2
            half(j, B0, B1)
            half(j + 1, B1, B0)
            return carry
        lax.fori_loop(0, ch // 2, pair, 0)

        wait_gathers(B0)
        wait_idx(B1)
        wait_scatters(B1)
        wait_den(B1)
        plsc.subcore_barrier()
        pltpu.sync_copy(num_acc.at[pl.ds(row0, rows_per_sub)],
                        num_hbm.at[c, pl.ds(row0, rows_per_sub)])
        @pl.when(s == 0)
        def _():
            pltpu.sync_copy(den_acc, den_hbm.at[c])

    return edge_kernel


def _mm4(x, w1, w2, w3, w4):
    n = x.shape[0]
    blk = 1000
    grid = n // blk

    def body(x_ref, w1_ref, w2_ref, w3_ref, w4_ref, o1, o2, o3, o4):
        xb = x_ref[...]
        for w_ref, o_ref in ((w1_ref, o1), (w2_ref, o2),
                             (w3_ref, o3), (w4_ref, o4)):
            o_ref[...] = lax.dot_general(
                xb, w_ref[...], (((1,), (1,)), ((), ())),
                preferred_element_type=jnp.float32,
                precision=lax.Precision.HIGHEST)

    wspec = pl.BlockSpec((D, D), lambda i: (0, 0))
    return pl.pallas_call(
        body,
        grid=(grid,),
        in_specs=[pl.BlockSpec((blk, D), lambda i: (i, 0))] + [wspec] * 4,
        out_specs=[pl.BlockSpec((blk, D), lambda i: (i, 0))] * 4,
        out_shape=[jax.ShapeDtypeStruct((n, D), jnp.float32)] * 4,
    )(x, w1, w2, w3, w4)


def _combine(num_ref, den_ref, b_ref, g_ref, be_ref):
    a = num_ref[0]
    den = den_ref[:, 0:1]
    for i in range(1, NC):
        a = a + num_ref[i]
        den = den + den_ref[:, i:i + 1]
    h = a / (den + 1e-16) + b_ref[...]
    mean = jnp.mean(h, axis=0, keepdims=True)
    var = jnp.mean((h - mean) ** 2, axis=0, keepdims=True)
    return (h - mean) * lax.rsqrt(var + 1e-5) * g_ref[...] + be_ref[...]


def _mid(num, den, b, g, be, wl, wr):
    n = num.shape[1]

    def body(num_ref, den_ref, b_ref, g_ref, be_ref, wl_ref, wr_ref,
             ol, orr):
        hn = jnp.maximum(_combine(num_ref, den_ref, b_ref, g_ref, be_ref),
                         0.0)
        for w_ref, o_ref in ((wl_ref, ol), (wr_ref, orr)):
            o_ref[...] = lax.dot_general(
                hn, w_ref[...], (((1,), (1,)), ((), ())),
                preferred_element_type=jnp.float32,
                precision=lax.Precision.HIGHEST)

    return pl.pallas_call(
        body,
        out_shape=[jax.ShapeDtypeStruct((n, D), jnp.float32)] * 2,
    )(num, den, b, g, be, wl, wr)


def _final(num2, den2, b2, g2, be2, num3, den3, b3, g3, be3):
    n = num2.shape[1]

    def body(n2, d2, b2r, g2r, be2r, n3, d3, b3r, g3r, be3r, out):
        o2 = _combine(n2, d2, b2r, g2r, be2r)
        o3 = _combine(n3, d3, b3r, g3r, be3r)
        out[...] = jnp.maximum(o2 + o3, 0.0)

    return pl.pallas_call(
        body,
        out_shape=jax.ShapeDtypeStruct((n, D), jnp.float32),
    )(num2, den2, b2, g2, be2, num3, den3, b3, g3, be3)


def kernel(x, edge_index, Wl1, Wr1, att1, b1, g1, be1,
           Wl2, Wr2, att2, b2, g2, be2,
           Wl3, Wr3, att3, b3, g3, be3):
    n = x.shape[0]
    e = edge_index.shape[1]
    e_total = e + n
    ch = -(-e_total // (NC * NS * CHUNK))
    ch = ch + (ch % 2)
    e_pad = NC * NS * CHUNK * ch + 2 * CHUNK

    idt = edge_index.dtype
    loop = jnp.arange(n, dtype=idt)
    padz = jnp.zeros((e_pad - e_total,), idt)
    src = jnp.concatenate([edge_index[0], loop, padz]).astype(jnp.int32)
    dst = jnp.concatenate([edge_index[1], loop, padz]).astype(jnp.int32)

    npad = -(-n // (NS * 8)) * (NS * 8)
    edge_pass = _make_edge_pass(n, npad, e_total, ch)

    def split(acc):
        num, den = acc
        return num[:, :n], den.reshape(NC, npad)[:, :n].T

    def row(v):
        return v.reshape(1, D)

    xl1, xr1, xl3, xr3 = _mm4(x, Wl1, Wr1, Wl3, Wr3)
    num1, den1 = split(edge_pass(xl1, xr1, src, dst, att1))
    xl2, xr2 = _mid(num1, den1, row(b1), row(g1), row(be1), Wl2, Wr2)
    num2, den2 = split(edge_pass(xl2, xr2, src, dst, att2))
    num3, den3 = split(edge_pass(xl3, xr3, src, dst, att3))
    return _final(num2, den2, row(b2), row(g2), row(be2),
                  num3, den3, row(b3), row(g3), row(be3))
